# initial kernel scaffold (unmeasured)
import jax
import jax.numpy as jnp
from jax import lax
from jax.experimental import pallas as pl
from jax.experimental.pallas import tpu as pltpu

N_DEV = 32
B, SQ, SKV, D, DH = 4, 256, 1024, 1024, 128
HQ = 8
HKV = 2
GROUP = HQ // HKV
ROWS = B * SQ
CHUNK = ROWS // N_DEV
SCALE = 0.08838834764831843


def _body(x_ref, wq_ref, k_ref, v_ref, wo_ref, out_ref,
          attn_ref, rs_buf, rs_send, rs_recv, ag_send, ag_recv):
    my = lax.axis_index("i")
    left = lax.rem(my + N_DEV - 1, N_DEV)
    right = lax.rem(my + 1, N_DEV)

    barrier = pltpu.get_barrier_semaphore()
    for nbr in (left, right):
        pl.semaphore_signal(barrier, inc=1, device_id=(nbr,),
                            device_id_type=pl.DeviceIdType.MESH)
    pl.semaphore_wait(barrier, 2)

    q2d = jnp.dot(x_ref[...], wq_ref[...], preferred_element_type=jnp.float32)
    for b in range(B):
        for h in range(HQ):
            kvi = b * HKV + h // GROUP
            qh = q2d[b * SQ:(b + 1) * SQ, h * DH:(h + 1) * DH]
            s = jnp.dot(qh, k_ref[kvi],
                        preferred_element_type=jnp.float32) * SCALE
            m = jnp.max(s, axis=1, keepdims=True)
            p = jnp.exp(s - m)
            l = jnp.sum(p, axis=1, keepdims=True)
            o = jnp.dot(p, v_ref[kvi], preferred_element_type=jnp.float32) / l
            attn_ref[b * SQ:(b + 1) * SQ, h * DH:(h + 1) * DH] = o
    out_ref[...] = jnp.dot(attn_ref[...], wo_ref[...],
                           preferred_element_type=jnp.float32)

    for st in range(N_DEV - 1):
        src_c = lax.rem(my - st + N_DEV, N_DEV)
        rdma = pltpu.make_async_remote_copy(
            src_ref=out_ref.at[pl.ds(src_c * CHUNK, CHUNK), :],
            dst_ref=rs_buf.at[st],
            send_sem=rs_send.at[st],
            recv_sem=rs_recv.at[st],
            device_id=(right,),
            device_id_type=pl.DeviceIdType.MESH,
        )
        rdma.start()
        rdma.wait()
        rc = lax.rem(my - 1 - st + 2 * N_DEV, N_DEV)
        sl = pl.ds(rc * CHUNK, CHUNK)
        out_ref[sl, :] = out_ref[sl, :] + rs_buf[st]

    for st in range(N_DEV - 1):
        src_c = lax.rem(my + 1 - st + 2 * N_DEV, N_DEV)
        sl = pl.ds(src_c * CHUNK, CHUNK)
        rdma = pltpu.make_async_remote_copy(
            src_ref=out_ref.at[sl, :],
            dst_ref=out_ref.at[sl, :],
            send_sem=ag_send.at[st],
            recv_sem=ag_recv.at[st],
            device_id=(right,),
            device_id_type=pl.DeviceIdType.MESH,
        )
        rdma.start()
        rdma.wait()


def kernel(x, Wq, Wo, K_ext, V_ext):
    i = lax.axis_index("i")
    x2d = x.reshape(ROWS, D)
    K_sl = lax.dynamic_slice_in_dim(K_ext, HKV * i, HKV, axis=2)
    V_sl = lax.dynamic_slice_in_dim(V_ext, HKV * i, HKV, axis=2)
    Kt = K_sl.transpose(0, 2, 3, 1).reshape(B * HKV, DH, SKV)
    Vt = V_sl.transpose(0, 2, 1, 3).reshape(B * HKV, SKV, DH)

    out2d = pl.pallas_call(
        _body,
        out_shape=jax.ShapeDtypeStruct((ROWS, D), jnp.float32),
        in_specs=[pl.BlockSpec(memory_space=pltpu.VMEM)] * 5,
        out_specs=pl.BlockSpec(memory_space=pltpu.VMEM),
        scratch_shapes=[
            pltpu.VMEM((ROWS, D), jnp.float32),
            pltpu.VMEM((N_DEV - 1, CHUNK, D), jnp.float32),
            pltpu.SemaphoreType.DMA((N_DEV - 1,)),
            pltpu.SemaphoreType.DMA((N_DEV - 1,)),
            pltpu.SemaphoreType.DMA((N_DEV - 1,)),
            pltpu.SemaphoreType.DMA((N_DEV - 1,)),
        ],
        compiler_params=pltpu.CompilerParams(
            collective_id=0, vmem_limit_bytes=100 * 1024 * 1024,
        ),
    )(x2d, Wq, Kt, Vt, Wo)
    return out2d.reshape(B, SQ, D)


# baseline (device time: 139686 ns/iter reference)
import jax
import jax.numpy as jnp
from jax import lax
from jax.experimental import pallas as pl
from jax.experimental.pallas import tpu as pltpu

N_DEV = 32
B, SQ, SKV, D, DH = 4, 256, 1024, 1024, 128
HQ = 8
HKV = 2
GROUP = HQ // HKV
ROWS = B * SQ
CHUNK = ROWS // N_DEV
SCALE = 0.08838834764831843


BITS = ((0, 3, 1, 2, 4), (3, 0, 2, 1, 4))
HALF = (512, 256, 128, 64, 32)
RSOFF = (0, 512, 768, 896, 960)
CSPLIT = D // 2


def _body(x_ref, wq_ref, k_ref, v_ref, wo_ref, out_ref,
          attn_ref, rs_buf, rs_send, rs_recv, ag_send, ag_recv):
    my = lax.axis_index("i")

    barrier = pltpu.get_barrier_semaphore()
    for b in range(5):
        pl.semaphore_signal(barrier, inc=1, device_id=(my ^ (1 << b),),
                            device_id_type=pl.DeviceIdType.MESH)
    pl.semaphore_wait(barrier, 5)

    q2d = jnp.dot(x_ref[...], wq_ref[...], preferred_element_type=jnp.float32)
    for b in range(B):
        for h in range(HQ):
            kvi = b * HKV + h // GROUP
            qh = q2d[b * SQ:(b + 1) * SQ, h * DH:(h + 1) * DH]
            s = jnp.dot(qh, k_ref[kvi],
                        preferred_element_type=jnp.float32) * SCALE
            m = jnp.max(s, axis=1, keepdims=True)
            p = jnp.exp(s - m)
            l = jnp.sum(p, axis=1, keepdims=True)
            o = jnp.dot(p, v_ref[kvi], preferred_element_type=jnp.float32) / l
            attn_ref[b * SQ:(b + 1) * SQ, h * DH:(h + 1) * DH] = o
    out_ref[...] = jnp.dot(attn_ref[...], wo_ref[...],
                           preferred_element_type=jnp.float32)

    bases = [my * 0, my * 0]
    for s in range(5):
        half = HALF[s]
        rdmas = []
        for q in range(2):
            bit = BITS[q][s]
            mybit = (my >> bit) & 1
            partner = my ^ (1 << bit)
            send_row = bases[q] + (1 - mybit) * half
            rdma = pltpu.make_async_remote_copy(
                src_ref=out_ref.at[pl.ds(send_row, half), pl.ds(q * CSPLIT, CSPLIT)],
                dst_ref=rs_buf.at[pl.ds(RSOFF[s], half), pl.ds(q * CSPLIT, CSPLIT)],
                send_sem=rs_send.at[q * 5 + s],
                recv_sem=rs_recv.at[q * 5 + s],
                device_id=(partner,),
                device_id_type=pl.DeviceIdType.MESH,
            )
            rdma.start()
            rdmas.append(rdma)
            bases[q] = bases[q] + mybit * half
        for q in range(2):
            rdmas[q].wait()
            sl = (pl.ds(bases[q], half), pl.ds(q * CSPLIT, CSPLIT))
            out_ref[sl] = out_ref[sl] + rs_buf[pl.ds(RSOFF[s], half),
                                               pl.ds(q * CSPLIT, CSPLIT)]

    for s in range(4, -1, -1):
        half = HALF[s]
        rdmas = []
        for q in range(2):
            bit = BITS[q][s]
            partner = my ^ (1 << bit)
            sl = (pl.ds(bases[q], half), pl.ds(q * CSPLIT, CSPLIT))
            rdma = pltpu.make_async_remote_copy(
                src_ref=out_ref.at[sl],
                dst_ref=out_ref.at[sl],
                send_sem=ag_send.at[q * 5 + s],
                recv_sem=ag_recv.at[q * 5 + s],
                device_id=(partner,),
                device_id_type=pl.DeviceIdType.MESH,
            )
            rdma.start()
            rdmas.append(rdma)
        for q in range(2):
            rdmas[q].wait()
            bit = BITS[q][s]
            mybit = (my >> bit) & 1
            bases[q] = bases[q] - mybit * half


def kernel(x, Wq, Wo, K_ext, V_ext):
    i = lax.axis_index("i")
    x2d = x.reshape(ROWS, D)
    K_sl = lax.dynamic_slice_in_dim(K_ext, HKV * i, HKV, axis=2)
    V_sl = lax.dynamic_slice_in_dim(V_ext, HKV * i, HKV, axis=2)
    Kt = K_sl.transpose(0, 2, 3, 1).reshape(B * HKV, DH, SKV)
    Vt = V_sl.transpose(0, 2, 1, 3).reshape(B * HKV, SKV, DH)

    out2d = pl.pallas_call(
        _body,
        out_shape=jax.ShapeDtypeStruct((ROWS, D), jnp.float32),
        in_specs=[pl.BlockSpec(memory_space=pltpu.VMEM)] * 5,
        out_specs=pl.BlockSpec(memory_space=pltpu.VMEM),
        scratch_shapes=[
            pltpu.VMEM((ROWS, D), jnp.float32),
            pltpu.VMEM((992, D), jnp.float32),
            pltpu.SemaphoreType.DMA((10,)),
            pltpu.SemaphoreType.DMA((10,)),
            pltpu.SemaphoreType.DMA((10,)),
            pltpu.SemaphoreType.DMA((10,)),
        ],
        compiler_params=pltpu.CompilerParams(
            collective_id=0, vmem_limit_bytes=100 * 1024 * 1024,
        ),
    )(x2d, Wq, Kt, Vt, Wo)
    return out2d.reshape(B, SQ, D)


# device time: 135731 ns/iter; 1.0291x vs baseline; 1.0291x over previous
import jax
import jax.numpy as jnp
from jax import lax
from jax.experimental import pallas as pl
from jax.experimental.pallas import tpu as pltpu

N_DEV = 32
B, SQ, SKV, D, DH = 4, 256, 1024, 1024, 128
HQ = 8
HKV = 2
GROUP = HQ // HKV
ROWS = B * SQ
CHUNK = ROWS // N_DEV
SCALE = 0.08838834764831843


BITS = ((0, 3, 1, 2, 4), (3, 0, 2, 1, 4))
HALF = (512, 256, 128, 64, 32)
RSOFF = (0, 512, 768, 896, 960)
CSPLIT = D // 2


def _body(x_ref, wq_ref, k_ref, v_ref, wo_ref, out_ref,
          attn_ref, rs_buf, rs_send, rs_recv, ag_send, ag_recv):
    my = lax.axis_index("i")

    barrier = pltpu.get_barrier_semaphore()
    for b in range(5):
        pl.semaphore_signal(barrier, inc=1, device_id=(my ^ (1 << b),),
                            device_id_type=pl.DeviceIdType.MESH)
    pl.semaphore_wait(barrier, 5)

    q2d = jnp.dot(x_ref[...], wq_ref[...], preferred_element_type=jnp.float32)
    for b in range(B):
        for h in range(HQ):
            kvi = b * HKV + h // GROUP
            qh = q2d[b * SQ:(b + 1) * SQ, h * DH:(h + 1) * DH]
            s = jnp.dot(qh, k_ref[kvi],
                        preferred_element_type=jnp.float32) * SCALE
            m = jnp.max(s, axis=1, keepdims=True)
            p = jnp.exp(s - m)
            l = jnp.sum(p, axis=1, keepdims=True)
            o = jnp.dot(p, v_ref[kvi], preferred_element_type=jnp.float32) / l
            attn_ref[b * SQ:(b + 1) * SQ, h * DH:(h + 1) * DH] = o

    bases = [my * 0, my * 0]
    rdmas = [None, None]

    def rs_start(q, s):
        bit = BITS[q][s]
        mybit = (my >> bit) & 1
        half = HALF[s]
        send_row = bases[q] + (1 - mybit) * half
        rdma = pltpu.make_async_remote_copy(
            src_ref=out_ref.at[pl.ds(send_row, half), pl.ds(q * CSPLIT, CSPLIT)],
            dst_ref=rs_buf.at[pl.ds(RSOFF[s], half), pl.ds(q * CSPLIT, CSPLIT)],
            send_sem=rs_send.at[q * 5 + s],
            recv_sem=rs_recv.at[q * 5 + s],
            device_id=(my ^ (1 << bit),),
            device_id_type=pl.DeviceIdType.MESH,
        )
        rdma.start()
        rdmas[q] = rdma
        bases[q] = bases[q] + mybit * half

    def rs_finish(q, s):
        half = HALF[s]
        rdmas[q].wait()
        sl = (pl.ds(bases[q], half), pl.ds(q * CSPLIT, CSPLIT))
        out_ref[sl] = out_ref[sl] + rs_buf[pl.ds(RSOFF[s], half),
                                           pl.ds(q * CSPLIT, CSPLIT)]

    def ag_start(q, s):
        bit = BITS[q][s]
        half = HALF[s]
        sl = (pl.ds(bases[q], half), pl.ds(q * CSPLIT, CSPLIT))
        rdma = pltpu.make_async_remote_copy(
            src_ref=out_ref.at[sl],
            dst_ref=out_ref.at[sl],
            send_sem=ag_send.at[q * 5 + s],
            recv_sem=ag_recv.at[q * 5 + s],
            device_id=(my ^ (1 << bit),),
            device_id_type=pl.DeviceIdType.MESH,
        )
        rdma.start()
        rdmas[q] = rdma

    def ag_finish(q, s):
        rdmas[q].wait()
        mybit = (my >> BITS[q][s]) & 1
        bases[q] = bases[q] - mybit * HALF[s]

    out_ref[:, 0:CSPLIT] = jnp.dot(attn_ref[...], wo_ref[:, 0:CSPLIT],
                                   preferred_element_type=jnp.float32)
    rs_start(0, 0)
    out_ref[:, CSPLIT:D] = jnp.dot(attn_ref[...], wo_ref[:, CSPLIT:D],
                                   preferred_element_type=jnp.float32)
    rs_start(1, 0)
    for s in range(4):
        rs_finish(0, s)
        rs_start(0, s + 1)
        rs_finish(1, s)
        rs_start(1, s + 1)
    rs_finish(0, 4)
    ag_start(0, 4)
    rs_finish(1, 4)
    ag_start(1, 4)
    for s in range(4, 0, -1):
        ag_finish(0, s)
        ag_start(0, s - 1)
        ag_finish(1, s)
        ag_start(1, s - 1)
    ag_finish(0, 0)
    ag_finish(1, 0)


def kernel(x, Wq, Wo, K_ext, V_ext):
    i = lax.axis_index("i")
    x2d = x.reshape(ROWS, D)
    K_sl = lax.dynamic_slice_in_dim(K_ext, HKV * i, HKV, axis=2)
    V_sl = lax.dynamic_slice_in_dim(V_ext, HKV * i, HKV, axis=2)
    Kt = K_sl.transpose(0, 2, 3, 1).reshape(B * HKV, DH, SKV)
    Vt = V_sl.transpose(0, 2, 1, 3).reshape(B * HKV, SKV, DH)

    out2d = pl.pallas_call(
        _body,
        out_shape=jax.ShapeDtypeStruct((ROWS, D), jnp.float32),
        in_specs=[pl.BlockSpec(memory_space=pltpu.VMEM)] * 5,
        out_specs=pl.BlockSpec(memory_space=pltpu.VMEM),
        scratch_shapes=[
            pltpu.VMEM((ROWS, D), jnp.float32),
            pltpu.VMEM((992, D), jnp.float32),
            pltpu.SemaphoreType.DMA((10,)),
            pltpu.SemaphoreType.DMA((10,)),
            pltpu.SemaphoreType.DMA((10,)),
            pltpu.SemaphoreType.DMA((10,)),
        ],
        compiler_params=pltpu.CompilerParams(
            collective_id=0, vmem_limit_bytes=100 * 1024 * 1024,
        ),
    )(x2d, Wq, Kt, Vt, Wo)
    return out2d.reshape(B, SQ, D)
